# Initial kernel scaffold; baseline (speedup 1.0000x reference)
import jax
import jax.numpy as jnp
from jax.experimental import pallas as pl


def kernel(x, edge_index, W1a, b1a, W1b, b1b, W2a, b2a, W2b, b2b):
    raise NotImplementedError("write your pallas kernel here")


# speedup vs baseline: 2.4097x; 2.4097x over previous
import functools

import jax
import jax.numpy as jnp
from jax import lax
from jax.experimental import pallas as pl
from jax.experimental.pallas import tpu as pltpu
from jax.experimental.pallas import tpu_sc as plsc

N_NODES = 10000
N_EDGES = 160000
IN_F = 256
HID_F = 1024

CHUNK_W = 128
N_TILES = 16
EDGES_PER_TILE = N_EDGES // N_TILES
EDGE_BLK = 80
N_BLKS = EDGES_PER_TILE // EDGE_BLK
ACC_ROWS = 10240
ROWS_PER_TILE = ACC_ROWS // N_TILES
DRAIN_BLK = 128
N_DRAIN = ROWS_PER_TILE // DRAIN_BLK


def _make_sc_agg(n_chunks):
    n_pass = n_chunks // 2
    mesh = plsc.VectorSubcoreMesh(core_axis_name="c", subcore_axis_name="s")

    @functools.partial(
        pl.kernel,
        out_type=jax.ShapeDtypeStruct((n_chunks, ACC_ROWS, CHUNK_W), jnp.float32),
        mesh=mesh,
        scratch_types=[
            pltpu.VMEM_SHARED((ACC_ROWS, CHUNK_W), jnp.float32),
            pltpu.VMEM((EDGE_BLK,), jnp.int32),
            pltpu.VMEM((EDGE_BLK,), jnp.int32),
            pltpu.VMEM((EDGE_BLK, CHUNK_W), jnp.float32),
            pltpu.VMEM((DRAIN_BLK, CHUNK_W), jnp.float32),
            pltpu.VMEM((DRAIN_BLK, CHUNK_W), jnp.float32),
            pltpu.SemaphoreType.DMA,
        ],
    )
    def sc_agg(*refs):
        xs = refs[:n_chunks]
        src_hbm, dst_hbm, out_hbm = refs[n_chunks:n_chunks + 3]
        agg_sh, src_v, dst_v, rows_v, zero_v, drain_v, sem = refs[n_chunks + 3:]
        c = lax.axis_index("c")
        s = lax.axis_index("s")

        def _zi(i, _):
            for k in range(CHUNK_W // 16):
                zero_v[i, pl.ds(k * 16, 16)] = jnp.zeros((16,), jnp.float32)
            return 0
        lax.fori_loop(0, DRAIN_BLK, _zi, 0)

        def one_pass(x_hbm, chunk):
            for j in range(N_DRAIN):
                pltpu.sync_copy(
                    zero_v, agg_sh.at[pl.ds(s * ROWS_PER_TILE + j * DRAIN_BLK, DRAIN_BLK)])
            plsc.subcore_barrier()

            def blk(b, _):
                base = s * EDGES_PER_TILE + b * EDGE_BLK
                pltpu.sync_copy(src_hbm.at[pl.ds(base, EDGE_BLK)], src_v)
                pltpu.async_copy(x_hbm.at[src_v], rows_v, sem).wait()
                pltpu.sync_copy(dst_hbm.at[pl.ds(base, EDGE_BLK)], dst_v)
                pltpu.sync_copy(rows_v, agg_sh.at[dst_v], add=True)
                return 0
            lax.fori_loop(0, N_BLKS, blk, 0)
            plsc.subcore_barrier()

            for j in range(N_DRAIN):
                row0 = s * ROWS_PER_TILE + j * DRAIN_BLK
                pltpu.sync_copy(agg_sh.at[pl.ds(row0, DRAIN_BLK)], drain_v)
                pltpu.sync_copy(drain_v, out_hbm.at[chunk, pl.ds(row0, DRAIN_BLK)])
            plsc.subcore_barrier()

        for p in range(n_pass):
            for ccode in range(2):
                @pl.when(c == ccode)
                def _(p=p, ccode=ccode):
                    one_pass(xs[2 * p + ccode], 2 * p + ccode)

    return sc_agg


_sc_agg_2 = _make_sc_agg(2)
_sc_agg_8 = _make_sc_agg(8)


def _mlp1_body(x_ref, agg_ref, wa_ref, ba_ref, wb_ref, bb_ref, h_ref):
    xin = x_ref[...] + jnp.concatenate([agg_ref[0], agg_ref[1]], axis=-1)
    t = jnp.dot(xin, wa_ref[...], preferred_element_type=jnp.float32) + ba_ref[...]
    t = jnp.maximum(t, 0.0)
    h = jnp.dot(t, wb_ref[...], preferred_element_type=jnp.float32) + bb_ref[...]
    h_ref[...] = jnp.maximum(h, 0.0)


def _mlp2_body(h_ref, agg_ref, wa_ref, ba_ref, wb_ref, bb_ref, o_ref):
    i = pl.program_id(0)
    zin = h_ref[...] + jnp.concatenate(
        [agg_ref[j] for j in range(HID_F // CHUNK_W)], axis=-1)
    t = jnp.dot(zin, wa_ref[...], preferred_element_type=jnp.float32) + ba_ref[...]
    t = jnp.maximum(t, 0.0)
    r = jnp.dot(t, wb_ref[...], preferred_element_type=jnp.float32) + bb_ref[...]
    r = jnp.maximum(r, 0.0)
    part = jnp.sum(r, axis=0, keepdims=True)

    @pl.when(i == 0)
    def _():
        o_ref[...] = part

    @pl.when(i != 0)
    def _():
        o_ref[...] = o_ref[...] + part


ROW_BLK = 1000
N_ROW_BLKS = N_NODES // ROW_BLK


def _mlp1(x, agg1, W1a, b1a, W1b, b1b):
    return pl.pallas_call(
        _mlp1_body,
        grid=(N_ROW_BLKS,),
        in_specs=[
            pl.BlockSpec((ROW_BLK, IN_F), lambda i: (i, 0)),
            pl.BlockSpec((2, ROW_BLK, CHUNK_W), lambda i: (0, i, 0)),
            pl.BlockSpec((IN_F, HID_F), lambda i: (0, 0)),
            pl.BlockSpec((1, HID_F), lambda i: (0, 0)),
            pl.BlockSpec((HID_F, HID_F), lambda i: (0, 0)),
            pl.BlockSpec((1, HID_F), lambda i: (0, 0)),
        ],
        out_specs=pl.BlockSpec((ROW_BLK, HID_F), lambda i: (i, 0)),
        out_shape=jax.ShapeDtypeStruct((N_NODES, HID_F), jnp.float32),
    )(x, agg1, W1a, b1a.reshape(1, -1), W1b, b1b.reshape(1, -1))


def _mlp2(h, agg2, W2a, b2a, W2b, b2b):
    out = pl.pallas_call(
        _mlp2_body,
        grid=(N_ROW_BLKS,),
        in_specs=[
            pl.BlockSpec((ROW_BLK, HID_F), lambda i: (i, 0)),
            pl.BlockSpec((HID_F // CHUNK_W, ROW_BLK, CHUNK_W), lambda i: (0, i, 0)),
            pl.BlockSpec((HID_F, HID_F), lambda i: (0, 0)),
            pl.BlockSpec((1, HID_F), lambda i: (0, 0)),
            pl.BlockSpec((HID_F, IN_F), lambda i: (0, 0)),
            pl.BlockSpec((1, IN_F), lambda i: (0, 0)),
        ],
        out_specs=pl.BlockSpec((1, IN_F), lambda i: (0, 0)),
        out_shape=jax.ShapeDtypeStruct((1, IN_F), jnp.float32),
    )(h, agg2, W2a, b2a.reshape(1, -1), W2b, b2b.reshape(1, -1))
    return out.reshape(IN_F)


def kernel(x, edge_index, W1a, b1a, W1b, b1b, W2a, b2a, W2b, b2b):
    src = edge_index[0].astype(jnp.int32)
    dst = edge_index[1].astype(jnp.int32)

    x_chunks = tuple(x[:, i * CHUNK_W:(i + 1) * CHUNK_W]
                     for i in range(IN_F // CHUNK_W))
    agg1 = _sc_agg_2(*x_chunks, src, dst)[:, :N_NODES]

    h = _mlp1(x, agg1, W1a, b1a, W1b, b1b)

    h_chunks = tuple(h[:, i * CHUNK_W:(i + 1) * CHUNK_W]
                     for i in range(HID_F // CHUNK_W))
    agg2 = _sc_agg_8(*h_chunks, src, dst)[:, :N_NODES]

    return _mlp2(h, agg2, W2a, b2a, W2b, b2b)


# speedup vs baseline: 2.7961x; 1.1603x over previous
import functools

import jax
import jax.numpy as jnp
from jax import lax
from jax.experimental import pallas as pl
from jax.experimental.pallas import tpu as pltpu
from jax.experimental.pallas import tpu_sc as plsc

N_NODES = 10000
N_EDGES = 160000
IN_F = 256
HID_F = 1024

CHUNK_W = 128
N_TILES = 16
EDGES_PER_TILE = 10240
EDGE_BLK = 80
N_BLKS = EDGES_PER_TILE // EDGE_BLK
ACC_ROWS = 10240
ROWS_PER_TILE = ACC_ROWS // N_TILES
DRAIN_BLK = 32
N_DRAIN = ROWS_PER_TILE // DRAIN_BLK


def _make_sc_agg(n_chunks):
    n_pass = n_chunks // 2
    mesh = plsc.VectorSubcoreMesh(core_axis_name="c", subcore_axis_name="s")

    @functools.partial(
        pl.kernel,
        out_type=jax.ShapeDtypeStruct((n_chunks, ACC_ROWS, CHUNK_W), jnp.float32),
        mesh=mesh,
        scratch_types=[
            pltpu.VMEM_SHARED((ACC_ROWS, CHUNK_W), jnp.float32),
            pltpu.VMEM((EDGES_PER_TILE,), jnp.int32),
            pltpu.VMEM((EDGE_BLK,), jnp.int32),
            pltpu.VMEM((EDGE_BLK,), jnp.int32),
            pltpu.VMEM((EDGE_BLK, CHUNK_W), jnp.float32),
            pltpu.VMEM((EDGE_BLK, CHUNK_W), jnp.float32),
            pltpu.VMEM((DRAIN_BLK, CHUNK_W), jnp.float32),
            pltpu.VMEM((DRAIN_BLK, CHUNK_W), jnp.float32),
            pltpu.SemaphoreType.DMA,
            pltpu.SemaphoreType.DMA,
            pltpu.SemaphoreType.DMA,
            pltpu.SemaphoreType.DMA,
        ],
    )
    def sc_agg(*refs):
        xs = refs[:n_chunks]
        src_hbm, dst_hbm, out_hbm = refs[n_chunks:n_chunks + 3]
        (agg_sh, src_all, d0, d1, rows0, rows1, zero_v, drain_v,
         sem0, sem1, semd0, semd1) = refs[n_chunks + 3:]
        c = lax.axis_index("c")
        s = lax.axis_index("s")
        dbuf = (d0, d1)
        rbuf = (rows0, rows1)
        gsem = (sem0, sem1)
        dsem = (semd0, semd1)

        pltpu.sync_copy(src_hbm.at[pl.ds(s * EDGES_PER_TILE, EDGES_PER_TILE)],
                        src_all)

        def _zi(i, _):
            for k in range(CHUNK_W // 16):
                zero_v[i, pl.ds(k * 16, 16)] = jnp.zeros((16,), jnp.float32)
            return 0
        lax.fori_loop(0, DRAIN_BLK, _zi, 0)

        def one_pass(x_hbm, chunk):
            for j in range(N_DRAIN):
                pltpu.sync_copy(
                    zero_v, agg_sh.at[pl.ds(s * ROWS_PER_TILE + j * DRAIN_BLK, DRAIN_BLK)])
            plsc.subcore_barrier()

            def start_b(b, k):
                pltpu.async_copy(
                    dst_hbm.at[pl.ds(s * EDGES_PER_TILE + b * EDGE_BLK, EDGE_BLK)],
                    dbuf[k], dsem[k])
                pltpu.async_copy(x_hbm.at[src_all.at[pl.ds(b * EDGE_BLK, EDGE_BLK)]],
                                 rbuf[k], gsem[k])

            def fin_b(b, k):
                pltpu.make_async_copy(
                    dst_hbm.at[pl.ds(s * EDGES_PER_TILE + b * EDGE_BLK, EDGE_BLK)],
                    dbuf[k], dsem[k]).wait()
                pltpu.make_async_copy(x_hbm.at[src_all.at[pl.ds(b * EDGE_BLK, EDGE_BLK)]],
                                      rbuf[k], gsem[k]).wait()
                pltpu.sync_copy(rbuf[k], agg_sh.at[dbuf[k]], add=True)

            start_b(0, 0)
            start_b(1, 1)

            def pair(j, _):
                b0 = 2 * j
                fin_b(b0, 0)
                start_b(b0 + 2, 0)
                fin_b(b0 + 1, 1)
                start_b(b0 + 3, 1)
                return 0
            lax.fori_loop(0, N_BLKS // 2 - 1, pair, 0)
            fin_b(N_BLKS - 2, 0)
            fin_b(N_BLKS - 1, 1)
            plsc.subcore_barrier()

            for j in range(N_DRAIN):
                row0 = s * ROWS_PER_TILE + j * DRAIN_BLK
                pltpu.sync_copy(agg_sh.at[pl.ds(row0, DRAIN_BLK)], drain_v)
                pltpu.sync_copy(drain_v, out_hbm.at[chunk, pl.ds(row0, DRAIN_BLK)])
            plsc.subcore_barrier()

        for p in range(n_pass):
            for ccode in range(2):
                @pl.when(c == ccode)
                def _(p=p, ccode=ccode):
                    one_pass(xs[2 * p + ccode], 2 * p + ccode)

    return sc_agg


_sc_agg_2 = _make_sc_agg(2)
_sc_agg_8 = _make_sc_agg(8)


def _mlp1_body(x_ref, agg_ref, wa_ref, ba_ref, wb_ref, bb_ref, h_ref):
    xin = x_ref[...] + jnp.concatenate([agg_ref[0], agg_ref[1]], axis=-1)
    t = jnp.dot(xin, wa_ref[...], preferred_element_type=jnp.float32) + ba_ref[...]
    t = jnp.maximum(t, 0.0)
    h = jnp.dot(t, wb_ref[...], preferred_element_type=jnp.float32) + bb_ref[...]
    h_ref[...] = jnp.maximum(h, 0.0)


def _mlp2_body(h_ref, agg_ref, wa_ref, ba_ref, wb_ref, bb_ref, o_ref):
    i = pl.program_id(0)
    zin = h_ref[...] + jnp.concatenate(
        [agg_ref[j] for j in range(HID_F // CHUNK_W)], axis=-1)
    t = jnp.dot(zin, wa_ref[...], preferred_element_type=jnp.float32) + ba_ref[...]
    t = jnp.maximum(t, 0.0)
    r = jnp.dot(t, wb_ref[...], preferred_element_type=jnp.float32) + bb_ref[...]
    r = jnp.maximum(r, 0.0)
    part = jnp.sum(r, axis=0, keepdims=True)

    @pl.when(i == 0)
    def _():
        o_ref[...] = part

    @pl.when(i != 0)
    def _():
        o_ref[...] = o_ref[...] + part


ROW_BLK = 1000
N_ROW_BLKS = N_NODES // ROW_BLK


def _mlp1(x, agg1, W1a, b1a, W1b, b1b):
    return pl.pallas_call(
        _mlp1_body,
        grid=(N_ROW_BLKS,),
        in_specs=[
            pl.BlockSpec((ROW_BLK, IN_F), lambda i: (i, 0)),
            pl.BlockSpec((2, ROW_BLK, CHUNK_W), lambda i: (0, i, 0)),
            pl.BlockSpec((IN_F, HID_F), lambda i: (0, 0)),
            pl.BlockSpec((1, HID_F), lambda i: (0, 0)),
            pl.BlockSpec((HID_F, HID_F), lambda i: (0, 0)),
            pl.BlockSpec((1, HID_F), lambda i: (0, 0)),
        ],
        out_specs=pl.BlockSpec((ROW_BLK, HID_F), lambda i: (i, 0)),
        out_shape=jax.ShapeDtypeStruct((N_NODES, HID_F), jnp.float32),
    )(x, agg1, W1a, b1a.reshape(1, -1), W1b, b1b.reshape(1, -1))


def _mlp2(h, agg2, W2a, b2a, W2b, b2b):
    out = pl.pallas_call(
        _mlp2_body,
        grid=(N_ROW_BLKS,),
        in_specs=[
            pl.BlockSpec((ROW_BLK, HID_F), lambda i: (i, 0)),
            pl.BlockSpec((HID_F // CHUNK_W, ROW_BLK, CHUNK_W), lambda i: (0, i, 0)),
            pl.BlockSpec((HID_F, HID_F), lambda i: (0, 0)),
            pl.BlockSpec((1, HID_F), lambda i: (0, 0)),
            pl.BlockSpec((HID_F, IN_F), lambda i: (0, 0)),
            pl.BlockSpec((1, IN_F), lambda i: (0, 0)),
        ],
        out_specs=pl.BlockSpec((1, IN_F), lambda i: (0, 0)),
        out_shape=jax.ShapeDtypeStruct((1, IN_F), jnp.float32),
    )(h, agg2, W2a, b2a.reshape(1, -1), W2b, b2b.reshape(1, -1))
    return out.reshape(IN_F)


def kernel(x, edge_index, W1a, b1a, W1b, b1b, W2a, b2a, W2b, b2b):
    e = edge_index.astype(jnp.int32)
    pad = EDGES_PER_TILE - N_EDGES // N_TILES
    src = jnp.pad(e[0].reshape(N_TILES, -1), ((0, 0), (0, pad)),
                  constant_values=0).reshape(-1)
    dst = jnp.pad(e[1].reshape(N_TILES, -1), ((0, 0), (0, pad)),
                  constant_values=ACC_ROWS - 1).reshape(-1)

    x_chunks = tuple(x[:, i * CHUNK_W:(i + 1) * CHUNK_W]
                     for i in range(IN_F // CHUNK_W))
    agg1 = _sc_agg_2(*x_chunks, src, dst)[:, :N_NODES]

    h = _mlp1(x, agg1, W1a, b1a, W1b, b1b)

    h_chunks = tuple(h[:, i * CHUNK_W:(i + 1) * CHUNK_W]
                     for i in range(HID_F // CHUNK_W))
    agg2 = _sc_agg_8(*h_chunks, src, dst)[:, :N_NODES]

    return _mlp2(h, agg2, W2a, b2a, W2b, b2b)
